# baseline (speedup 1.0000x reference)
import functools

import jax
import jax.numpy as jnp
from jax import lax
from jax.experimental import pallas as pl
from jax.experimental.pallas import tpu as pltpu
from jax.experimental.pallas import tpu_sc as plsc

N = 10000
E = 320000
D = 128

NC = 2
NS = 16
NW = NC * NS

K = 128
CH = 80
TOTCH = NW * CH
EP = TOTCH * K
NP = 10240
RPT = NP // NS

CNT0 = 80
CNT1 = CH * 2 - CNT0
CNTMAX = max(CNT0, CNT1)

_MESH = plsc.VectorSubcoreMesh(core_axis_name="c", subcore_axis_name="s")


@functools.partial(
    pl.kernel,
    out_type=jax.ShapeDtypeStruct((NC, NP), jnp.float32),
    mesh=_MESH,
    scratch_types=[
        pltpu.VMEM((CH, K), jnp.int32),
        pltpu.VMEM((K,), jnp.float32),
        pltpu.VMEM_SHARED((NP,), jnp.float32),
    ],
)
def _sc_degree(dst2, ones_h, z1d, out, dst_v, ones_v, acc_sh):
    cid = lax.axis_index("c")
    sid = lax.axis_index("s")
    wid = cid * NS + sid
    row0 = pl.multiple_of(sid * RPT, RPT)
    pltpu.sync_copy(z1d, acc_sh.at[pl.ds(row0, RPT)])
    pltpu.sync_copy(dst2.at[pl.ds(wid * CH, CH)], dst_v)
    pltpu.sync_copy(ones_h, ones_v)
    plsc.subcore_barrier()

    def body(j, carry):
        pltpu.sync_copy(ones_v, acc_sh.at[dst_v.at[j]], add=True)
        return carry

    lax.fori_loop(0, CH, body, 0)
    plsc.subcore_barrier()
    pltpu.sync_copy(acc_sh.at[pl.ds(row0, RPT)], out.at[cid, pl.ds(row0, RPT)])


@functools.partial(
    pl.kernel,
    out_type=jax.ShapeDtypeStruct((NC, NP, D), jnp.float32),
    mesh=_MESH,
    scratch_types=[
        pltpu.VMEM((CNTMAX, K), jnp.int32),
        pltpu.VMEM((CNTMAX, K), jnp.int32),
        pltpu.VMEM((K, D), jnp.float32),
        pltpu.VMEM_SHARED((NP, D), jnp.float32),
        pltpu.SemaphoreType.DMA,
    ],
)
def _sc_aggregate(table, src2, dst2, zblk, out, src_v, dst_v, rows_v, acc_sh,
                  gsem):
    cid = lax.axis_index("c")
    sid = lax.axis_index("s")
    row0 = pl.multiple_of(sid * RPT, RPT)
    pltpu.sync_copy(zblk, acc_sh.at[pl.ds(row0, RPT)])

    def run(start, cnt):
        pltpu.sync_copy(src2.at[pl.ds(start, cnt)], src_v.at[pl.ds(0, cnt)])
        pltpu.sync_copy(dst2.at[pl.ds(start, cnt)], dst_v.at[pl.ds(0, cnt)])
        plsc.subcore_barrier()

        def body(j, carry):
            pltpu.async_copy(table.at[src_v.at[j]], rows_v, gsem).wait()
            pltpu.sync_copy(rows_v, acc_sh.at[dst_v.at[j]], add=True)
            return carry

        lax.fori_loop(0, cnt, body, 0)

    @pl.when(cid == 0)
    def _core0():
        run(sid * CNT0, CNT0)

    @pl.when(cid == 1)
    def _core1():
        run(NS * CNT0 + sid * CNT1, CNT1)

    plsc.subcore_barrier()
    pltpu.sync_copy(acc_sh.at[pl.ds(row0, RPT)], out.at[cid, pl.ds(row0, RPT)])


_R = 2048


def _dinv_of(deg_ref):
    deg = deg_ref[:, 0:1] + deg_ref[:, 1:2] + 1.0
    return jnp.where(deg > 0, lax.rsqrt(deg), 0.0)


def _tc1_body(x_ref, w_ref, deg_ref, out_ref):
    dinv = _dinv_of(deg_ref)
    h = jnp.dot(x_ref[...], w_ref[...], preferred_element_type=jnp.float32)
    out_ref[...] = h * dinv


def _tc1(x_pad, W1, deg_t):
    return pl.pallas_call(
        _tc1_body,
        grid=(NP // _R,),
        in_specs=[
            pl.BlockSpec((_R, D), lambda i: (i, 0)),
            pl.BlockSpec((D, D), lambda i: (0, 0)),
            pl.BlockSpec((_R, NC), lambda i: (i, 0)),
        ],
        out_specs=pl.BlockSpec((_R, D), lambda i: (i, 0)),
        out_shape=jax.ShapeDtypeStruct((NP, D), jnp.float32),
    )(x_pad, W1, deg_t)


def _tc2_body(accp_ref, t1_ref, deg_ref, w_ref, b_ref, out_ref):
    dinv = _dinv_of(deg_ref)
    acc = accp_ref[0] + accp_ref[1]
    o1 = dinv * (acc + t1_ref[...]) + b_ref[...]
    g = jnp.maximum(o1, 0.0)
    h2 = jnp.dot(g, w_ref[...], preferred_element_type=jnp.float32)
    out_ref[...] = h2 * dinv


def _tc2(acc1, table1, deg_t, W2, b1):
    return pl.pallas_call(
        _tc2_body,
        grid=(NP // _R,),
        in_specs=[
            pl.BlockSpec((NC, _R, D), lambda i: (0, i, 0)),
            pl.BlockSpec((_R, D), lambda i: (i, 0)),
            pl.BlockSpec((_R, NC), lambda i: (i, 0)),
            pl.BlockSpec((D, D), lambda i: (0, 0)),
            pl.BlockSpec((1, D), lambda i: (0, 0)),
        ],
        out_specs=pl.BlockSpec((_R, D), lambda i: (i, 0)),
        out_shape=jax.ShapeDtypeStruct((NP, D), jnp.float32),
    )(acc1, table1, deg_t, W2, b1)


def _tc3_body(accp_ref, t2_ref, deg_ref, b_ref, out_ref):
    dinv = _dinv_of(deg_ref)
    acc = accp_ref[0] + accp_ref[1]
    out_ref[...] = dinv * (acc + t2_ref[...]) + b_ref[...]


def _tc3(acc2, table2, deg_t, b2):
    return pl.pallas_call(
        _tc3_body,
        grid=(NP // _R,),
        in_specs=[
            pl.BlockSpec((NC, _R, D), lambda i: (0, i, 0)),
            pl.BlockSpec((_R, D), lambda i: (i, 0)),
            pl.BlockSpec((_R, NC), lambda i: (i, 0)),
            pl.BlockSpec((1, D), lambda i: (0, 0)),
        ],
        out_specs=pl.BlockSpec((_R, D), lambda i: (i, 0)),
        out_shape=jax.ShapeDtypeStruct((NP, D), jnp.float32),
    )(acc2, table2, deg_t, b2)


def kernel(x, edge_index, W1, b1, W2, b2):
    pad = EP - E
    padv = jnp.full((pad,), N, dtype=jnp.int32)
    src2 = jnp.concatenate([edge_index[0], padv]).reshape(TOTCH, K)
    dst2 = jnp.concatenate([edge_index[1], padv]).reshape(TOTCH, K)
    x_pad = jnp.concatenate(
        [x, jnp.zeros((NP - N, D), dtype=jnp.float32)], axis=0)
    zblk = jnp.zeros((RPT, D), dtype=jnp.float32)
    z1d = jnp.zeros((RPT,), dtype=jnp.float32)
    ones_h = jnp.ones((K,), dtype=jnp.float32)

    degp = _sc_degree(dst2, ones_h, z1d)
    deg_t = degp.T

    table1 = _tc1(x_pad, W1, deg_t)
    acc1 = _sc_aggregate(table1, src2, dst2, zblk)
    table2 = _tc2(acc1, table1, deg_t, W2, b1.reshape(1, D))
    acc2 = _sc_aggregate(table2, src2, dst2, zblk)
    out = _tc3(acc2, table2, deg_t, b2.reshape(1, D))
    return out[:N]


# speedup vs baseline: 1.0002x; 1.0002x over previous
import functools

import jax
import jax.numpy as jnp
from jax import lax
from jax.experimental import pallas as pl
from jax.experimental.pallas import tpu as pltpu
from jax.experimental.pallas import tpu_sc as plsc

N = 10000
E = 320000
D = 128

NC = 2
NS = 16
NW = NC * NS

K = 128
CH = 80
TOTCH = NW * CH
EP = TOTCH * K
NP = 10240
RPT = NP // NS

CNT0 = 80
CNT1 = CH * 2 - CNT0
CNTMAX = max(CNT0, CNT1)

_MESH = plsc.VectorSubcoreMesh(core_axis_name="c", subcore_axis_name="s")


@functools.partial(
    pl.kernel,
    out_type=jax.ShapeDtypeStruct((NC, NP), jnp.float32),
    mesh=_MESH,
    scratch_types=[
        pltpu.VMEM((CH, K), jnp.int32),
        pltpu.VMEM((K,), jnp.float32),
        pltpu.VMEM_SHARED((NP,), jnp.float32),
    ],
)
def _sc_degree(dst2, ones_h, z1d, out, dst_v, ones_v, acc_sh):
    cid = lax.axis_index("c")
    sid = lax.axis_index("s")
    wid = cid * NS + sid
    row0 = pl.multiple_of(sid * RPT, RPT)
    pltpu.sync_copy(z1d, acc_sh.at[pl.ds(row0, RPT)])
    pltpu.sync_copy(dst2.at[pl.ds(wid * CH, CH)], dst_v)
    pltpu.sync_copy(ones_h, ones_v)
    plsc.subcore_barrier()

    def body(j, carry):
        pltpu.sync_copy(ones_v, acc_sh.at[dst_v.at[j]], add=True)
        return carry

    lax.fori_loop(0, CH, body, 0)
    plsc.subcore_barrier()
    pltpu.sync_copy(acc_sh.at[pl.ds(row0, RPT)], out.at[cid, pl.ds(row0, RPT)])


@functools.partial(
    pl.kernel,
    out_type=jax.ShapeDtypeStruct((NC, NP, D), jnp.float32),
    mesh=_MESH,
    scratch_types=[
        pltpu.VMEM((CH, K), jnp.int32),
        pltpu.VMEM((CH, K), jnp.int32),
        pltpu.VMEM((K, D), jnp.float32),
        pltpu.VMEM_SHARED((NP, D), jnp.float32),
        pltpu.SemaphoreType.DMA,
    ],
)
def _sc_aggregate(table, src2, dst2, zblk, out, src_v, dst_v, rows_v, acc_sh,
                  gsem):
    cid = lax.axis_index("c")
    sid = lax.axis_index("s")
    wid = cid * NS + sid
    row0 = pl.multiple_of(sid * RPT, RPT)
    pltpu.sync_copy(zblk, acc_sh.at[pl.ds(row0, RPT)])
    start = pl.multiple_of(wid * CH, CH)
    pltpu.sync_copy(src2.at[pl.ds(start, CH)], src_v)
    pltpu.sync_copy(dst2.at[pl.ds(start, CH)], dst_v)
    plsc.subcore_barrier()

    def body(j, carry):
        pltpu.async_copy(table.at[src_v.at[j]], rows_v, gsem).wait()
        pltpu.sync_copy(rows_v, acc_sh.at[dst_v.at[j]], add=True)
        return carry

    lax.fori_loop(0, CH, body, 0)
    plsc.subcore_barrier()
    pltpu.sync_copy(acc_sh.at[pl.ds(row0, RPT)], out.at[cid, pl.ds(row0, RPT)])


_R = 2048


def _dinv_of(deg_ref):
    deg = deg_ref[:, 0:1] + deg_ref[:, 1:2] + 1.0
    return jnp.where(deg > 0, lax.rsqrt(deg), 0.0)


def _tc1_body(x_ref, w_ref, deg_ref, out_ref):
    dinv = _dinv_of(deg_ref)
    h = jnp.dot(x_ref[...], w_ref[...], preferred_element_type=jnp.float32)
    out_ref[...] = h * dinv


def _tc1(x_pad, W1, deg_t):
    return pl.pallas_call(
        _tc1_body,
        grid=(NP // _R,),
        in_specs=[
            pl.BlockSpec((_R, D), lambda i: (i, 0)),
            pl.BlockSpec((D, D), lambda i: (0, 0)),
            pl.BlockSpec((_R, NC), lambda i: (i, 0)),
        ],
        out_specs=pl.BlockSpec((_R, D), lambda i: (i, 0)),
        out_shape=jax.ShapeDtypeStruct((NP, D), jnp.float32),
    )(x_pad, W1, deg_t)


def _tc2_body(accp_ref, t1_ref, deg_ref, w_ref, b_ref, out_ref):
    dinv = _dinv_of(deg_ref)
    acc = accp_ref[0] + accp_ref[1]
    o1 = dinv * (acc + t1_ref[...]) + b_ref[...]
    g = jnp.maximum(o1, 0.0)
    h2 = jnp.dot(g, w_ref[...], preferred_element_type=jnp.float32)
    out_ref[...] = h2 * dinv


def _tc2(acc1, table1, deg_t, W2, b1):
    return pl.pallas_call(
        _tc2_body,
        grid=(NP // _R,),
        in_specs=[
            pl.BlockSpec((NC, _R, D), lambda i: (0, i, 0)),
            pl.BlockSpec((_R, D), lambda i: (i, 0)),
            pl.BlockSpec((_R, NC), lambda i: (i, 0)),
            pl.BlockSpec((D, D), lambda i: (0, 0)),
            pl.BlockSpec((1, D), lambda i: (0, 0)),
        ],
        out_specs=pl.BlockSpec((_R, D), lambda i: (i, 0)),
        out_shape=jax.ShapeDtypeStruct((NP, D), jnp.float32),
    )(acc1, table1, deg_t, W2, b1)


def _tc3_body(accp_ref, t2_ref, deg_ref, b_ref, out_ref):
    dinv = _dinv_of(deg_ref)
    acc = accp_ref[0] + accp_ref[1]
    out_ref[...] = dinv * (acc + t2_ref[...]) + b_ref[...]


def _tc3(acc2, table2, deg_t, b2):
    return pl.pallas_call(
        _tc3_body,
        grid=(NP // _R,),
        in_specs=[
            pl.BlockSpec((NC, _R, D), lambda i: (0, i, 0)),
            pl.BlockSpec((_R, D), lambda i: (i, 0)),
            pl.BlockSpec((_R, NC), lambda i: (i, 0)),
            pl.BlockSpec((1, D), lambda i: (0, 0)),
        ],
        out_specs=pl.BlockSpec((_R, D), lambda i: (i, 0)),
        out_shape=jax.ShapeDtypeStruct((NP, D), jnp.float32),
    )(acc2, table2, deg_t, b2)


def kernel(x, edge_index, W1, b1, W2, b2):
    pad = EP - E
    padv = jnp.full((pad,), N, dtype=jnp.int32)
    src2 = jnp.concatenate([edge_index[0], padv]).reshape(TOTCH, K)
    dst2 = jnp.concatenate([edge_index[1], padv]).reshape(TOTCH, K)
    x_pad = jnp.concatenate(
        [x, jnp.zeros((NP - N, D), dtype=jnp.float32)], axis=0)
    zblk = jnp.zeros((RPT, D), dtype=jnp.float32)
    z1d = jnp.zeros((RPT,), dtype=jnp.float32)
    ones_h = jnp.ones((K,), dtype=jnp.float32)

    degp = _sc_degree(dst2, ones_h, z1d)
    deg_t = degp.T

    table1 = _tc1(x_pad, W1, deg_t)
    acc1 = _sc_aggregate(table1, src2, dst2, zblk)
    table2 = _tc2(acc1, table1, deg_t, W2, b1.reshape(1, D))
    acc2 = _sc_aggregate(table2, src2, dst2, zblk)
    out = _tc3(acc2, table2, deg_t, b2.reshape(1, D))
    return out[:N]


# speedup vs baseline: 1.2144x; 1.2141x over previous
import functools

import jax
import jax.numpy as jnp
from jax import lax
from jax.experimental import pallas as pl
from jax.experimental.pallas import tpu as pltpu
from jax.experimental.pallas import tpu_sc as plsc

N = 10000
E = 320000
D = 128

NC = 2
NS = 16
NW = NC * NS

K = 128
CH = 80
TOTCH = NW * CH
EP = TOTCH * K
NP = 10240
RPT = NP // NS

CNT0 = 80
CNT1 = CH * 2 - CNT0
CNTMAX = max(CNT0, CNT1)

_MESH = plsc.VectorSubcoreMesh(core_axis_name="c", subcore_axis_name="s")


@functools.partial(
    pl.kernel,
    out_type=jax.ShapeDtypeStruct((NC, NP), jnp.float32),
    mesh=_MESH,
    scratch_types=[
        pltpu.VMEM((CH, K), jnp.int32),
        pltpu.VMEM((K,), jnp.float32),
        pltpu.VMEM_SHARED((NP,), jnp.float32),
    ],
)
def _sc_degree(dst2, ones_h, z1d, out, dst_v, ones_v, acc_sh):
    cid = lax.axis_index("c")
    sid = lax.axis_index("s")
    wid = cid * NS + sid
    row0 = pl.multiple_of(sid * RPT, RPT)
    pltpu.sync_copy(z1d, acc_sh.at[pl.ds(row0, RPT)])
    pltpu.sync_copy(dst2.at[pl.ds(wid * CH, CH)], dst_v)
    pltpu.sync_copy(ones_h, ones_v)
    plsc.subcore_barrier()

    def body(j, carry):
        pltpu.sync_copy(ones_v, acc_sh.at[dst_v.at[j]], add=True)
        return carry

    lax.fori_loop(0, CH, body, 0)
    plsc.subcore_barrier()
    pltpu.sync_copy(acc_sh.at[pl.ds(row0, RPT)], out.at[cid, pl.ds(row0, RPT)])


@functools.partial(
    pl.kernel,
    out_type=jax.ShapeDtypeStruct((NC, NP, D), jnp.float32),
    mesh=_MESH,
    scratch_types=[
        pltpu.VMEM((CH, K), jnp.int32),
        pltpu.VMEM((CH // 2, K), jnp.int32),
        pltpu.VMEM((2, K), jnp.int32),
        pltpu.VMEM((2, K, D), jnp.float32),
        pltpu.VMEM_SHARED((NP, D), jnp.float32),
        pltpu.SemaphoreType.DMA,
        pltpu.SemaphoreType.DMA,
    ],
)
def _sc_aggregate(table, src2, dstp, zblk, out, src_v, dstp_v, dstu_v, rows_v,
                  acc_sh, gsem0, gsem1):
    cid = lax.axis_index("c")
    sid = lax.axis_index("s")
    wid = cid * NS + sid
    row0 = pl.multiple_of(sid * RPT, RPT)
    pltpu.sync_copy(zblk, acc_sh.at[pl.ds(row0, RPT)])
    pltpu.sync_copy(src2.at[pl.ds(pl.multiple_of(wid * CH, CH), CH)], src_v)
    pltpu.sync_copy(
        dstp.at[pl.ds(pl.multiple_of(wid * (CH // 2), CH // 2), CH // 2)],
        dstp_v)
    plsc.subcore_barrier()

    gsems = (gsem0, gsem1)
    pltpu.async_copy(table.at[src_v.at[0]], rows_v.at[0], gsems[0])

    def body(j0, carry):
        for g in range(8):
            v = dstp_v[j0, pl.ds(16 * g, 16)]
            lo = lax.bitwise_and(v, jnp.int32(0xFFFF))
            hi = lax.shift_right_logical(v, jnp.int32(16))
            dstu_v[g // 4, pl.ds((32 * g) % K, 16)] = lo
            dstu_v[g // 4, pl.ds((32 * g + 16) % K, 16)] = hi
        for b in range(2):
            j = 2 * j0 + b
            pltpu.make_async_copy(
                table.at[src_v.at[j]], rows_v.at[b], gsems[b]).wait()

            @pl.when(j + 1 < CH)
            def _fire():
                pltpu.async_copy(
                    table.at[src_v.at[j + 1]], rows_v.at[1 - b], gsems[1 - b])

            pltpu.sync_copy(rows_v.at[b], acc_sh.at[dstu_v.at[b]], add=True)
        return carry

    lax.fori_loop(0, CH // 2, body, 0)
    plsc.subcore_barrier()
    pltpu.sync_copy(acc_sh.at[pl.ds(row0, RPT)], out.at[cid, pl.ds(row0, RPT)])


_R = 2048


def _dinv_of(deg_ref):
    deg = deg_ref[:, 0:1] + deg_ref[:, 1:2] + 1.0
    return jnp.where(deg > 0, lax.rsqrt(deg), 0.0)


def _tc1_body(x_ref, w_ref, deg_ref, out_ref):
    dinv = _dinv_of(deg_ref)
    h = jnp.dot(x_ref[...], w_ref[...], preferred_element_type=jnp.float32)
    out_ref[...] = h * dinv


def _tc1(x_pad, W1, deg_t):
    return pl.pallas_call(
        _tc1_body,
        grid=(NP // _R,),
        in_specs=[
            pl.BlockSpec((_R, D), lambda i: (i, 0)),
            pl.BlockSpec((D, D), lambda i: (0, 0)),
            pl.BlockSpec((_R, NC), lambda i: (i, 0)),
        ],
        out_specs=pl.BlockSpec((_R, D), lambda i: (i, 0)),
        out_shape=jax.ShapeDtypeStruct((NP, D), jnp.float32),
    )(x_pad, W1, deg_t)


def _tc2_body(accp_ref, t1_ref, deg_ref, w_ref, b_ref, out_ref):
    dinv = _dinv_of(deg_ref)
    acc = accp_ref[0] + accp_ref[1]
    o1 = dinv * (acc + t1_ref[...]) + b_ref[...]
    g = jnp.maximum(o1, 0.0)
    h2 = jnp.dot(g, w_ref[...], preferred_element_type=jnp.float32)
    out_ref[...] = h2 * dinv


def _tc2(acc1, table1, deg_t, W2, b1):
    return pl.pallas_call(
        _tc2_body,
        grid=(NP // _R,),
        in_specs=[
            pl.BlockSpec((NC, _R, D), lambda i: (0, i, 0)),
            pl.BlockSpec((_R, D), lambda i: (i, 0)),
            pl.BlockSpec((_R, NC), lambda i: (i, 0)),
            pl.BlockSpec((D, D), lambda i: (0, 0)),
            pl.BlockSpec((1, D), lambda i: (0, 0)),
        ],
        out_specs=pl.BlockSpec((_R, D), lambda i: (i, 0)),
        out_shape=jax.ShapeDtypeStruct((NP, D), jnp.float32),
    )(acc1, table1, deg_t, W2, b1)


def _tc3_body(accp_ref, t2_ref, deg_ref, b_ref, out_ref):
    dinv = _dinv_of(deg_ref)
    acc = accp_ref[0] + accp_ref[1]
    out_ref[...] = dinv * (acc + t2_ref[...]) + b_ref[...]


def _tc3(acc2, table2, deg_t, b2):
    return pl.pallas_call(
        _tc3_body,
        grid=(NP // _R,),
        in_specs=[
            pl.BlockSpec((NC, _R, D), lambda i: (0, i, 0)),
            pl.BlockSpec((_R, D), lambda i: (i, 0)),
            pl.BlockSpec((_R, NC), lambda i: (i, 0)),
            pl.BlockSpec((1, D), lambda i: (0, 0)),
        ],
        out_specs=pl.BlockSpec((_R, D), lambda i: (i, 0)),
        out_shape=jax.ShapeDtypeStruct((NP, D), jnp.float32),
    )(acc2, table2, deg_t, b2)


def kernel(x, edge_index, W1, b1, W2, b2):
    pad = EP - E
    padv = jnp.full((pad,), N, dtype=jnp.int32)
    src2 = jnp.concatenate([edge_index[0], padv]).reshape(TOTCH, K)
    dflat = jnp.concatenate([edge_index[1], padv])
    dst2 = dflat.reshape(TOTCH, K)
    dr = dflat.reshape(EP // 32, 2, 16)
    dstp = jnp.bitwise_or(dr[:, 0, :],
                          jnp.left_shift(dr[:, 1, :], 16)).reshape(
                              TOTCH // 2, K)
    x_pad = jnp.concatenate(
        [x, jnp.zeros((NP - N, D), dtype=jnp.float32)], axis=0)
    zblk = jnp.zeros((RPT, D), dtype=jnp.float32)
    z1d = jnp.zeros((RPT,), dtype=jnp.float32)
    ones_h = jnp.ones((K,), dtype=jnp.float32)

    degp = _sc_degree(dst2, ones_h, z1d)
    deg_t = degp.T

    table1 = _tc1(x_pad, W1, deg_t)
    acc1 = _sc_aggregate(table1, src2, dstp, zblk)
    table2 = _tc2(acc1, table1, deg_t, W2, b1.reshape(1, D))
    acc2 = _sc_aggregate(table2, src2, dstp, zblk)
    out = _tc3(acc2, table2, deg_t, b2.reshape(1, D))
    return out[:N]
